# Initial kernel scaffold; baseline (speedup 1.0000x reference)
import jax
import jax.numpy as jnp
from jax.experimental import pallas as pl


def kernel(users_feat, bundles_feat, items_feat, ub_u, ub_b, ui_u, ui_i, bi_b, bi_i):
    raise NotImplementedError("write your pallas kernel here")


# speedup vs baseline: 11.6155x; 11.6155x over previous
import functools

import jax
import jax.numpy as jnp
from jax import lax
from jax.experimental import pallas as pl
from jax.experimental.pallas import tpu as pltpu
from jax.experimental.pallas import tpu_sc as plsc

_NU, _NB, _NI, _D = 50000, 20000, 100000, 64
_L = 2

_NC, _NS, _VL = 2, 16, 16
_NW = _NC * _NS
_GB = 128
_BB = 512
_GPB = _BB // _GB
_HC = 8
_R = 400


def _div_le(n, cap):
    for k in range(1, n + 1):
        if n % k == 0 and n // k <= cap:
            return n // k
    return 1


def _mesh():
    return plsc.VectorSubcoreMesh(core_axis_name="c", subcore_axis_name="s")


@functools.lru_cache(maxsize=None)
def _hist_fn(epad, nreal, npad):
    nbatch = epad // (_NW * _BB)
    sub = npad // _NS
    zb = _div_le(sub, 512)

    @functools.partial(
        pl.kernel,
        mesh=_mesh(),
        out_type=jax.ShapeDtypeStruct((_NC, npad, _HC), jnp.float32),
        scratch_types=[
            pltpu.VMEM((_BB,), jnp.int32),
            pltpu.VMEM((_GPB, _GB), jnp.int32),
            pltpu.VMEM((_GB, _HC), jnp.float32),
            pltpu.VMEM((zb, _HC), jnp.float32),
            pltpu.VMEM_SHARED((npad, _HC), jnp.float32),
            pltpu.SemaphoreType.DMA,
        ],
        compiler_params=pltpu.CompilerParams(use_tc_tiling_on_sc=False),
    )
    def hist(idx_hbm, ones_hbm, zeros_hbm, out_hbm,
             idx_v, lidx_v, ones_v, wb_v, cnt_sh, sem):
        del sem
        cid = lax.axis_index("c")
        sid = lax.axis_index("s")
        wid = sid * _NC + cid
        pltpu.sync_copy(ones_hbm, ones_v)
        pltpu.sync_copy(zeros_hbm, wb_v)
        for z in range(sub // zb):
            pltpu.sync_copy(wb_v, cnt_sh.at[pl.ds(sid * sub + z * zb, zb)])
        plsc.subcore_barrier()

        def batch(bi, c):
            off = (wid * nbatch + bi) * _BB
            pltpu.sync_copy(idx_hbm.at[pl.ds(off, _BB)], idx_v)
            for j in range(_BB // _VL):
                v = idx_v[pl.ds(j * _VL, _VL)]
                lv = jnp.where(v < nreal, v, nreal + (v & (_GB - 1)))
                lidx_v[j // (_GB // _VL), pl.ds((j % (_GB // _VL)) * _VL, _VL)] = lv
            for g in range(_GPB):
                pltpu.sync_copy(ones_v, cnt_sh.at[lidx_v.at[g]], add=True)
            return c

        lax.fori_loop(0, nbatch, batch, 0)
        plsc.subcore_barrier()
        for z in range(sub // zb):
            r0 = sid * sub + z * zb
            pltpu.sync_copy(cnt_sh.at[pl.ds(r0, zb)], wb_v)
            pltpu.sync_copy(wb_v, out_hbm.at[cid, pl.ds(r0, zb)])

    return hist


def _hist(dst_flat, nreal, epad):
    npad = -(-(nreal + _GB) // (_NS * _GB)) * (_NS * _GB)
    sub = npad // _NS
    zb = _div_le(sub, 512)
    ones = jnp.zeros((_GB, _HC), jnp.float32).at[:, 0].set(1.0)
    zeros = jnp.zeros((zb, _HC), jnp.float32)
    out = _hist_fn(epad, nreal, npad)(dst_flat, ones, zeros)
    return out[:, :nreal, 0]


_SB = 256
_SGP = _SB // _GB


@functools.lru_cache(maxsize=None)
def _spmm_fn(epad, chunk, nch):
    nbatch = epad // (_NS * _SB)
    ch_per_core = nch // _NC
    sub_rows = chunk // _NS
    wb = _div_le(sub_rows, _SB)

    @functools.partial(
        pl.kernel,
        mesh=_mesh(),
        out_type=jax.ShapeDtypeStruct((nch * chunk, _D), jnp.float32),
        scratch_types=[
            pltpu.VMEM((_SGP, _GB), jnp.int32),
            pltpu.VMEM((_SB,), jnp.int32),
            pltpu.VMEM((_SGP, _GB), jnp.int32),
            pltpu.VMEM((_SB, _D), jnp.float32),
            pltpu.VMEM_SHARED((chunk + _GB, _D), jnp.float32),
            pltpu.SemaphoreType.DMA,
        ],
        compiler_params=pltpu.CompilerParams(use_tc_tiling_on_sc=False),
    )
    def spmm(sidx_hbm, didx_hbm, feat_hbm, zeros_hbm, out_hbm,
             sidx_v, didx_v, lidx_v, rows_v, acc_sh, sem):
        cid = lax.axis_index("c")
        sid = lax.axis_index("s")
        for ci in range(ch_per_core):
            chunk_id = cid * ch_per_core + ci
            base = chunk_id * chunk
            pltpu.sync_copy(zeros_hbm, rows_v.at[pl.ds(0, _GB)])
            for z in range(sub_rows // _GB):
                pltpu.sync_copy(
                    rows_v.at[pl.ds(0, _GB)],
                    acc_sh.at[pl.ds(sid * sub_rows + z * _GB, _GB)])
            plsc.subcore_barrier()

            def batch(bi, c):
                row0 = (sid * nbatch + bi) * _SGP
                pltpu.sync_copy(sidx_hbm.at[pl.ds(row0, _SGP)], sidx_v)
                pltpu.sync_copy(didx_hbm.at[pl.ds(row0 * _GB, _SB)], didx_v)
                cps = [
                    pltpu.async_copy(feat_hbm.at[sidx_v.at[g]],
                                     rows_v.at[pl.ds(g * _GB, _GB)], sem)
                    for g in range(_SGP)
                ]
                for j in range(_SB // _VL):
                    v = didx_v[pl.ds(j * _VL, _VL)]
                    loc = v - base
                    ok = (loc >= 0) & (loc < chunk)
                    lv = jnp.where(ok, loc, chunk + (v & (_GB - 1)))
                    lidx_v[j // (_GB // _VL),
                           pl.ds((j % (_GB // _VL)) * _VL, _VL)] = lv
                for cp in cps:
                    cp.wait()
                for g in range(_SGP):
                    pltpu.sync_copy(rows_v.at[pl.ds(g * _GB, _GB)],
                                    acc_sh.at[lidx_v.at[g]], add=True)
                return c

            lax.fori_loop(0, nbatch, batch, 0)
            plsc.subcore_barrier()
            for w in range(sub_rows // wb):
                r0 = sid * sub_rows + w * wb
                pltpu.sync_copy(acc_sh.at[pl.ds(r0, wb)],
                                rows_v.at[pl.ds(0, wb)])
                pltpu.sync_copy(rows_v.at[pl.ds(0, wb)],
                                out_hbm.at[pl.ds(base + r0, wb)])
            if ci + 1 < ch_per_core:
                plsc.subcore_barrier()

    return spmm


def _chunk_cfg(nd):
    max_chunk = 26624
    nch = _NC * (-(-nd // (_NC * max_chunk)))
    chunk = -(-(-(-nd // nch)) // (_NS * _GB)) * (_NS * _GB)
    return chunk, nch


def _spmm(src2d, dst_flat, feat, nd, epad):
    chunk, nch = _chunk_cfg(nd)
    zeros = jnp.zeros((_GB, _D), jnp.float32)
    out = _spmm_fn(epad, chunk, nch)(src2d, dst_flat, feat, zeros)
    return out[:nd]


def _scale_body(x_ref, d_ref, o_ref, *, mode):
    d = d_ref[0, 0, 0, :] + d_ref[1, 0, 0, :]
    if mode == "rsqrt":
        s = 1.0 / (jnp.sqrt(d) + 1e-8)
    else:
        s = 1.0 / (d + 1e-8)
    o_ref[...] = x_ref[...] * s[:, None]


def _scale(x, deg2, mode):
    n = x.shape[0]
    return pl.pallas_call(
        functools.partial(_scale_body, mode=mode),
        grid=(n // _R,),
        in_specs=[
            pl.BlockSpec((_R, _D), lambda i: (i, 0)),
            pl.BlockSpec((_NC, 1, 1, _R), lambda i: (0, i, 0, 0)),
        ],
        out_specs=pl.BlockSpec((_R, _D), lambda i: (i, 0)),
        out_shape=jax.ShapeDtypeStruct((n, _D), jnp.float32),
    )(x, deg2.reshape(_NC, n // _R, 1, _R))


def _post_body(raw_ref, d_ref, acc_ref, scaled_ref, accout_ref, *, div):
    d = d_ref[0, 0, 0, :] + d_ref[1, 0, 0, :]
    inv = 1.0 / (jnp.sqrt(d) + 1e-8)
    cur = raw_ref[...] * inv[:, None] * (1.0 / div)
    scaled_ref[...] = cur * inv[:, None]
    nrm = jnp.maximum(jnp.sqrt(jnp.sum(cur * cur, axis=1, keepdims=True)),
                      1e-12)
    accout_ref[...] = acc_ref[...] + cur / nrm


def _post(raw, deg2, acc, div):
    n = raw.shape[0]
    return pl.pallas_call(
        functools.partial(_post_body, div=float(div)),
        grid=(n // _R,),
        in_specs=[
            pl.BlockSpec((_R, _D), lambda i: (i, 0)),
            pl.BlockSpec((_NC, 1, 1, _R), lambda i: (0, i, 0, 0)),
            pl.BlockSpec((_R, _D), lambda i: (i, 0)),
        ],
        out_specs=[pl.BlockSpec((_R, _D), lambda i: (i, 0))] * 2,
        out_shape=[jax.ShapeDtypeStruct((n, _D), jnp.float32)] * 2,
    )(raw, deg2.reshape(_NC, n // _R, 1, _R), acc)


def _combine_body(a_ref, b_ref, c_ref, o_ref):
    o_ref[...] = 0.5 * a_ref[...] + 0.2 * b_ref[...] + 0.3 * c_ref[...]


def _combine(a, b, c):
    n = a.shape[0]
    return pl.pallas_call(
        _combine_body,
        grid=(n // _R,),
        in_specs=[pl.BlockSpec((_R, _D), lambda i: (i, 0))] * 3,
        out_specs=pl.BlockSpec((_R, _D), lambda i: (i, 0)),
        out_shape=jax.ShapeDtypeStruct((n, _D), jnp.float32),
    )(a, b, c)


def _pad_edges(src, dst, nsrc, epad):
    e = src.shape[0]
    pad = epad - e
    psrc = jnp.concatenate(
        [src, jnp.arange(pad, dtype=jnp.int32) % nsrc])
    pdst = jnp.concatenate(
        [dst, jnp.full((pad,), 1 << 30, jnp.int32)])
    return psrc.reshape(epad // _GB, _GB), pdst


def _propagate(a, b, na, nb, af, bf):
    epad = -(-a.shape[0] // (_NW * _BB)) * (_NW * _BB)
    b_s, a_d = _pad_edges(b, a, nb, epad)
    a_s, b_d = _pad_edges(a, b, na, epad)
    deg2a = _hist(a_d, na, epad)
    deg2b = _hist(b_d, nb, epad)
    sa = _scale(af, deg2a, "rsqrt")
    sb = _scale(bf, deg2b, "rsqrt")
    acca, accb = af, bf
    for i in range(_L):
        rawa = _spmm(b_s, a_d, sb, na, epad)
        rawb = _spmm(a_s, b_d, sa, nb, epad)
        sa, acca = _post(rawa, deg2a, acca, i + 2)
        sb, accb = _post(rawb, deg2b, accb, i + 2)
    return acca, accb, (b_s, a_d, deg2a, epad)


def kernel(users_feat, bundles_feat, items_feat,
           ub_u, ub_b, ui_u, ui_i, bi_b, bi_i):
    ub_users, ub_bundles, _ = _propagate(
        ub_u, ub_b, _NU, _NB, users_feat, bundles_feat)
    ui_users, ui_items, (ui_is, ui_ud, ui_degu, ui_epad) = _propagate(
        ui_u, ui_i, _NU, _NI, users_feat, items_feat)
    bi_bundles, bi_items, (bi_is, bi_bd, bi_degb, bi_epad) = _propagate(
        bi_b, bi_i, _NB, _NI, bundles_feat, items_feat)

    ui_b_raw = _spmm(bi_is, bi_bd, items_feat, _NB, bi_epad)
    ui_bundles = _scale(ui_b_raw, bi_degb, "recip")
    bi_u_raw = _spmm(ui_is, ui_ud, bi_items, _NU, ui_epad)
    bi_users = _scale(bi_u_raw, ui_degu, "recip")

    users_rep = _combine(ub_users, ui_users, bi_users)
    bundles_rep = _combine(ub_bundles, ui_bundles, bi_bundles)
    return users_rep, bundles_rep


# speedup vs baseline: 17.7419x; 1.5274x over previous
import functools

import jax
import jax.numpy as jnp
from jax import lax
from jax.experimental import pallas as pl
from jax.experimental.pallas import tpu as pltpu
from jax.experimental.pallas import tpu_sc as plsc

_NU, _NB, _NI, _D = 50000, 20000, 100000, 64
_L = 2

_NC, _NS, _VL = 2, 16, 16
_NW = _NC * _NS
_GB = 128
_BB = 512
_GPB = _BB // _GB
_HC = 8
_R = 400


def _div_le(n, cap):
    for k in range(1, n + 1):
        if n % k == 0 and n // k <= cap:
            return n // k
    return 1


def _mesh():
    return plsc.VectorSubcoreMesh(core_axis_name="c", subcore_axis_name="s")


@functools.lru_cache(maxsize=None)
def _hist_fn(epad, nreal, npad):
    nbatch = epad // (_NW * _BB)
    sub = npad // _NS
    zb = _div_le(sub, 512)

    @functools.partial(
        pl.kernel,
        mesh=_mesh(),
        out_type=jax.ShapeDtypeStruct((_NC, npad, _HC), jnp.float32),
        scratch_types=[
            pltpu.VMEM((_BB,), jnp.int32),
            pltpu.VMEM((_GPB, _GB), jnp.int32),
            pltpu.VMEM((_GB, _HC), jnp.float32),
            pltpu.VMEM((zb, _HC), jnp.float32),
            pltpu.VMEM_SHARED((npad, _HC), jnp.float32),
            pltpu.SemaphoreType.DMA,
        ],
        compiler_params=pltpu.CompilerParams(use_tc_tiling_on_sc=False),
    )
    def hist(idx_hbm, ones_hbm, zeros_hbm, out_hbm,
             idx_v, lidx_v, ones_v, wb_v, cnt_sh, sem):
        del sem
        cid = lax.axis_index("c")
        sid = lax.axis_index("s")
        wid = sid * _NC + cid
        pltpu.sync_copy(ones_hbm, ones_v)
        pltpu.sync_copy(zeros_hbm, wb_v)
        for z in range(sub // zb):
            pltpu.sync_copy(wb_v, cnt_sh.at[pl.ds(sid * sub + z * zb, zb)])
        plsc.subcore_barrier()

        def batch(bi, c):
            off = (wid * nbatch + bi) * _BB
            pltpu.sync_copy(idx_hbm.at[pl.ds(off, _BB)], idx_v)
            for j in range(_BB // _VL):
                v = idx_v[pl.ds(j * _VL, _VL)]
                lv = jnp.where(v < nreal, v, nreal + (v & (_GB - 1)))
                lidx_v[j // (_GB // _VL), pl.ds((j % (_GB // _VL)) * _VL, _VL)] = lv
            for g in range(_GPB):
                pltpu.sync_copy(ones_v, cnt_sh.at[lidx_v.at[g]], add=True)
            return c

        lax.fori_loop(0, nbatch, batch, 0)
        plsc.subcore_barrier()
        for z in range(sub // zb):
            r0 = sid * sub + z * zb
            pltpu.sync_copy(cnt_sh.at[pl.ds(r0, zb)], wb_v)
            pltpu.sync_copy(wb_v, out_hbm.at[cid, pl.ds(r0, zb)])

    return hist


def _hist(dst_flat, nreal, epad):
    npad = -(-(nreal + _GB) // (_NS * _GB)) * (_NS * _GB)
    sub = npad // _NS
    zb = _div_le(sub, 512)
    ones = jnp.zeros((_GB, _HC), jnp.float32).at[:, 0].set(1.0)
    zeros = jnp.zeros((zb, _HC), jnp.float32)
    out = _hist_fn(epad, nreal, npad)(dst_flat, ones, zeros)
    return out[:, :nreal, 0]


_IC = 1024
_ICG = _IC // _GB


@functools.lru_cache(maxsize=None)
def _spmm_fn(epad, chunk, nch):
    nbatch = epad // (_NS * _IC)
    ch_per_core = nch // _NC
    sub_rows = chunk // _NS

    @functools.partial(
        pl.kernel,
        mesh=_mesh(),
        out_type=jax.ShapeDtypeStruct((nch * chunk, _D), jnp.float32),
        scratch_types=[
            pltpu.VMEM((_ICG, _GB), jnp.int32),
            pltpu.VMEM((_IC,), jnp.int32),
            pltpu.VMEM((_ICG, _GB), jnp.int32),
            pltpu.VMEM((2, _GB, _D), jnp.float32),
            pltpu.VMEM_SHARED((chunk + _GB, _D), jnp.float32),
            pltpu.SemaphoreType.DMA,
        ],
        compiler_params=pltpu.CompilerParams(use_tc_tiling_on_sc=False),
    )
    def spmm(sidx_hbm, didx_hbm, feat_hbm, zeros_hbm, out_hbm,
             sidx_v, didx_v, lidx_v, ring_v, acc_sh, sem):
        cid = lax.axis_index("c")
        sid = lax.axis_index("s")
        for ci in range(ch_per_core):
            chunk_id = cid * ch_per_core + ci
            base = chunk_id * chunk
            pltpu.sync_copy(zeros_hbm, ring_v.at[0])
            for z in range(sub_rows // _GB):
                pltpu.sync_copy(
                    ring_v.at[0],
                    acc_sh.at[pl.ds(sid * sub_rows + z * _GB, _GB)])
            plsc.subcore_barrier()

            def batch(bi, c):
                row0 = (sid * nbatch + bi) * _ICG
                pltpu.sync_copy(sidx_hbm.at[pl.ds(row0, _ICG)], sidx_v)
                pltpu.sync_copy(didx_hbm.at[pl.ds(row0 * _GB, _IC)], didx_v)
                cps = [pltpu.async_copy(feat_hbm.at[sidx_v.at[0]],
                                        ring_v.at[0], sem)]
                for j in range(_IC // _VL):
                    v = didx_v[pl.ds(j * _VL, _VL)]
                    loc = v - base
                    ok = (loc >= 0) & (loc < chunk)
                    lv = jnp.where(ok, loc, chunk + (v & (_GB - 1)))
                    lidx_v[j // (_GB // _VL),
                           pl.ds((j % (_GB // _VL)) * _VL, _VL)] = lv
                for g in range(_ICG):
                    if g + 1 < _ICG:
                        cps.append(
                            pltpu.async_copy(feat_hbm.at[sidx_v.at[g + 1]],
                                             ring_v.at[(g + 1) % 2], sem))
                    cps[g].wait()
                    pltpu.sync_copy(ring_v.at[g % 2],
                                    acc_sh.at[lidx_v.at[g]], add=True)
                return c

            lax.fori_loop(0, nbatch, batch, 0)
            plsc.subcore_barrier()
            for w in range(sub_rows // _GB):
                r0 = sid * sub_rows + w * _GB
                pltpu.sync_copy(acc_sh.at[pl.ds(r0, _GB)], ring_v.at[0])
                pltpu.sync_copy(ring_v.at[0],
                                out_hbm.at[pl.ds(base + r0, _GB)])
            if ci + 1 < ch_per_core:
                plsc.subcore_barrier()

    return spmm


def _chunk_cfg(nd):
    max_chunk = 26624
    nch = _NC * (-(-nd // (_NC * max_chunk)))
    chunk = -(-(-(-nd // nch)) // (_NS * _GB)) * (_NS * _GB)
    return chunk, nch


def _spmm(src2d, dst_flat, feat, nd, epad):
    chunk, nch = _chunk_cfg(nd)
    zeros = jnp.zeros((_GB, _D), jnp.float32)
    out = _spmm_fn(epad, chunk, nch)(src2d, dst_flat, feat, zeros)
    return out[:nd]


def _scale_body(x_ref, d_ref, o_ref, *, mode):
    d = d_ref[0, 0, 0, :] + d_ref[1, 0, 0, :]
    if mode == "rsqrt":
        s = 1.0 / (jnp.sqrt(d) + 1e-8)
    else:
        s = 1.0 / (d + 1e-8)
    o_ref[...] = x_ref[...] * s[:, None]


def _scale(x, deg2, mode):
    n = x.shape[0]
    return pl.pallas_call(
        functools.partial(_scale_body, mode=mode),
        grid=(n // _R,),
        in_specs=[
            pl.BlockSpec((_R, _D), lambda i: (i, 0)),
            pl.BlockSpec((_NC, 1, 1, _R), lambda i: (0, i, 0, 0)),
        ],
        out_specs=pl.BlockSpec((_R, _D), lambda i: (i, 0)),
        out_shape=jax.ShapeDtypeStruct((n, _D), jnp.float32),
    )(x, deg2.reshape(_NC, n // _R, 1, _R))


def _post_body(raw_ref, d_ref, acc_ref, scaled_ref, accout_ref, *, div):
    d = d_ref[0, 0, 0, :] + d_ref[1, 0, 0, :]
    inv = 1.0 / (jnp.sqrt(d) + 1e-8)
    cur = raw_ref[...] * inv[:, None] * (1.0 / div)
    scaled_ref[...] = cur * inv[:, None]
    nrm = jnp.maximum(jnp.sqrt(jnp.sum(cur * cur, axis=1, keepdims=True)),
                      1e-12)
    accout_ref[...] = acc_ref[...] + cur / nrm


def _post(raw, deg2, acc, div):
    n = raw.shape[0]
    return pl.pallas_call(
        functools.partial(_post_body, div=float(div)),
        grid=(n // _R,),
        in_specs=[
            pl.BlockSpec((_R, _D), lambda i: (i, 0)),
            pl.BlockSpec((_NC, 1, 1, _R), lambda i: (0, i, 0, 0)),
            pl.BlockSpec((_R, _D), lambda i: (i, 0)),
        ],
        out_specs=[pl.BlockSpec((_R, _D), lambda i: (i, 0))] * 2,
        out_shape=[jax.ShapeDtypeStruct((n, _D), jnp.float32)] * 2,
    )(raw, deg2.reshape(_NC, n // _R, 1, _R), acc)


def _combine_body(a_ref, b_ref, c_ref, o_ref):
    o_ref[...] = 0.5 * a_ref[...] + 0.2 * b_ref[...] + 0.3 * c_ref[...]


def _combine(a, b, c):
    n = a.shape[0]
    return pl.pallas_call(
        _combine_body,
        grid=(n // _R,),
        in_specs=[pl.BlockSpec((_R, _D), lambda i: (i, 0))] * 3,
        out_specs=pl.BlockSpec((_R, _D), lambda i: (i, 0)),
        out_shape=jax.ShapeDtypeStruct((n, _D), jnp.float32),
    )(a, b, c)


def _pad_edges(src, dst, nsrc, epad):
    e = src.shape[0]
    pad = epad - e
    psrc = jnp.concatenate(
        [src, jnp.arange(pad, dtype=jnp.int32) % nsrc])
    pdst = jnp.concatenate(
        [dst, jnp.full((pad,), 1 << 30, jnp.int32)])
    return psrc.reshape(epad // _GB, _GB), pdst


def _propagate(a, b, na, nb, af, bf):
    epad = -(-a.shape[0] // (_NW * _BB)) * (_NW * _BB)
    b_s, a_d = _pad_edges(b, a, nb, epad)
    a_s, b_d = _pad_edges(a, b, na, epad)
    deg2a = _hist(a_d, na, epad)
    deg2b = _hist(b_d, nb, epad)
    sa = _scale(af, deg2a, "rsqrt")
    sb = _scale(bf, deg2b, "rsqrt")
    acca, accb = af, bf
    for i in range(_L):
        rawa = _spmm(b_s, a_d, sb, na, epad)
        rawb = _spmm(a_s, b_d, sa, nb, epad)
        sa, acca = _post(rawa, deg2a, acca, i + 2)
        sb, accb = _post(rawb, deg2b, accb, i + 2)
    return acca, accb, (b_s, a_d, deg2a, epad)


def kernel(users_feat, bundles_feat, items_feat,
           ub_u, ub_b, ui_u, ui_i, bi_b, bi_i):
    ub_users, ub_bundles, _ = _propagate(
        ub_u, ub_b, _NU, _NB, users_feat, bundles_feat)
    ui_users, ui_items, (ui_is, ui_ud, ui_degu, ui_epad) = _propagate(
        ui_u, ui_i, _NU, _NI, users_feat, items_feat)
    bi_bundles, bi_items, (bi_is, bi_bd, bi_degb, bi_epad) = _propagate(
        bi_b, bi_i, _NB, _NI, bundles_feat, items_feat)

    ui_b_raw = _spmm(bi_is, bi_bd, items_feat, _NB, bi_epad)
    ui_bundles = _scale(ui_b_raw, bi_degb, "recip")
    bi_u_raw = _spmm(ui_is, ui_ud, bi_items, _NU, ui_epad)
    bi_users = _scale(bi_u_raw, ui_degu, "recip")

    users_rep = _combine(ub_users, ui_users, bi_users)
    bundles_rep = _combine(ub_bundles, ui_bundles, bi_bundles)
    return users_rep, bundles_rep
